# baseline (speedup 1.0000x reference)
import jax
import jax.numpy as jnp
from jax import lax
from jax.experimental import pallas as pl
from jax.experimental.pallas import tpu as pltpu
from jax.experimental.pallas import tpu_sc as plsc

N = 10000
E = 320000
F = 128
NUM_RELS = 8
NUM_BASES = 4

NPAD = 10240
NC = 2
NS = 16
NW = NC * NS
CH = 64
NU = 2
NCHUNK = 160
EPT = NCHUNK * CH
EPAD = EPT * NW
EPADT = 327680
ACC_ROWS = 10112
RPT = ACC_ROWS // NS
RBLK = 512
GBLK = EPADT // F // (NPAD // RBLK)


def _z_body(coeff_ref, x_ref, w_ref, src_ref, ety_ref, z_ref, g_ref):
    x = x_ref[...]
    ys = [
        jnp.dot(x, w_ref[b], preferred_element_type=jnp.float32)
        for b in range(NUM_BASES)
    ]
    for r in range(NUM_RELS):
        acc = ys[0] * coeff_ref[r, 0]
        for b in range(1, NUM_BASES):
            acc = acc + ys[b] * coeff_ref[r, b]
        z_ref[r] = acc
    g_ref[...] = ety_ref[...] * NPAD + src_ref[...]


def _compute_z(featp, w, coeff, src2d, ety2d):
    grid = NPAD // RBLK
    return pl.pallas_call(
        _z_body,
        grid=(grid,),
        in_specs=[
            pl.BlockSpec(memory_space=pltpu.SMEM),
            pl.BlockSpec((RBLK, F), lambda i: (i, 0)),
            pl.BlockSpec((NUM_BASES, F, F), lambda i: (0, 0, 0)),
            pl.BlockSpec((GBLK, F), lambda i: (i, 0)),
            pl.BlockSpec((GBLK, F), lambda i: (i, 0)),
        ],
        out_specs=[
            pl.BlockSpec((NUM_RELS, RBLK, F), lambda i: (0, i, 0)),
            pl.BlockSpec((GBLK, F), lambda i: (i, 0)),
        ],
        out_shape=[
            jax.ShapeDtypeStruct((NUM_RELS, NPAD, F), jnp.float32),
            jax.ShapeDtypeStruct((EPADT // F, F), jnp.int32),
        ],
    )(coeff, featp, w, src2d, ety2d)


def _sc_body(z_hbm, g_hbm, dst_hbm, zeros_hbm, out_hbm,
             g_v, dst_v, rows0, rows1, acc, sem0, sem1):
    cid = lax.axis_index("c")
    sid = lax.axis_index("s")
    wid = sid * NC + cid
    bufs = ((rows0, sem0), (rows1, sem1))

    pltpu.sync_copy(zeros_hbm.at[pl.ds(sid * RPT, RPT)],
                    acc.at[pl.ds(sid * RPT, RPT)])
    pltpu.sync_copy(g_hbm.at[wid], g_v)
    pltpu.sync_copy(dst_hbm.at[wid], dst_v)
    plsc.subcore_barrier()

    def gather_start(off, j):
        rows_v, sem = bufs[j]
        pltpu.async_copy(z_hbm.at[g_v.at[pl.ds(off, CH)]], rows_v, sem)

    def gather_wait(off, j):
        rows_v, sem = bufs[j]
        pltpu.make_async_copy(z_hbm.at[g_v.at[pl.ds(off, CH)]],
                              rows_v, sem).wait()

    def scatter(c, j):
        rows_v, _ = bufs[j]
        pltpu.sync_copy(rows_v, acc.at[dst_v.at[c]], add=True)

    for j in range(NU):
        gather_start(j * CH, j)

    def steady_body(k, _):
        c0 = k * NU
        off0 = pl.multiple_of(c0 * CH, NU * CH)
        for j in range(NU):
            gather_wait(off0 + j * CH, j)
            scatter(c0 + j, j)
            gather_start(off0 + (j + NU) * CH, j)
        return 0

    lax.fori_loop(0, NCHUNK // NU - 1, steady_body, 0)

    for j in range(NU):
        c = NCHUNK - NU + j
        gather_wait(c * CH, j)
        scatter(c, j)

    plsc.subcore_barrier()

    pltpu.sync_copy(acc.at[pl.ds(sid * RPT, RPT)],
                    out_hbm.at[cid, pl.ds(sid * RPT, RPT)])


def _sc_aggregate(zflat, gb, dstb, zeros):
    mesh = plsc.VectorSubcoreMesh(core_axis_name="c", subcore_axis_name="s")
    run = pl.kernel(
        _sc_body,
        mesh=mesh,
        out_type=jax.ShapeDtypeStruct((NC, NPAD, F), jnp.float32),
        scratch_types=[
            pltpu.VMEM((EPT,), jnp.int32),
            pltpu.VMEM((NCHUNK, CH), jnp.int32),
            pltpu.VMEM((CH, F), jnp.float32),
            pltpu.VMEM((CH, F), jnp.float32),
            pltpu.VMEM_SHARED((ACC_ROWS, F), jnp.float32),
            pltpu.SemaphoreType.DMA,
            pltpu.SemaphoreType.DMA,
        ],
    )
    return run(zflat, gb, dstb, zeros)


def _combine_body(a0_ref, a1_ref, x_ref, lw_ref, b_ref, o_ref):
    o_ref[...] = (
        a0_ref[0]
        + a1_ref[0]
        + jnp.dot(x_ref[...], lw_ref[...], preferred_element_type=jnp.float32)
        + b_ref[...]
    )


def _combine(accs, featp, loop_weight, h_bias):
    grid = NPAD // RBLK
    return pl.pallas_call(
        _combine_body,
        grid=(grid,),
        in_specs=[
            pl.BlockSpec((1, RBLK, F), lambda i: (0, i, 0)),
            pl.BlockSpec((1, RBLK, F), lambda i: (1, i, 0)),
            pl.BlockSpec((RBLK, F), lambda i: (i, 0)),
            pl.BlockSpec((F, F), lambda i: (0, 0)),
            pl.BlockSpec((1, F), lambda i: (0, 0)),
        ],
        out_specs=pl.BlockSpec((RBLK, F), lambda i: (i, 0)),
        out_shape=jax.ShapeDtypeStruct((NPAD, F), jnp.float32),
    )(accs, accs, featp, loop_weight, h_bias.reshape(1, F))


def kernel(feat, edge_index, edge_types, W, coeff, h_bias, loop_weight):
    src = edge_index[0].astype(jnp.int32)
    dst = edge_index[1].astype(jnp.int32)
    ety = edge_types.astype(jnp.int32)

    src_p = jnp.concatenate([src, jnp.zeros((EPADT - E,), jnp.int32)])
    ety_p = jnp.concatenate([ety, jnp.zeros((EPADT - E,), jnp.int32)])
    dst_p = jnp.concatenate(
        [dst, N + (jnp.arange(EPAD - E, dtype=jnp.int32) % (ACC_ROWS - N))])

    src2d = src_p.reshape(EPADT // F, F)
    ety2d = ety_p.reshape(EPADT // F, F)
    dstb = dst_p.reshape(NW, NCHUNK, CH)

    featp = jnp.pad(feat, ((0, NPAD - N), (0, 0)))
    zeros = jnp.zeros((ACC_ROWS, F), jnp.float32)

    z, g = _compute_z(featp, W, coeff, src2d, ety2d)
    zflat = z.reshape(NUM_RELS * NPAD, F)
    gb = g.reshape(EPADT)[:EPAD].reshape(NW, EPT)
    accs = _sc_aggregate(zflat, gb, dstb, zeros)
    h = _combine(accs, featp, loop_weight, h_bias)
    return h[:N]
